# baseline (speedup 1.0000x reference)
import functools

import jax
import jax.numpy as jnp
from jax import lax
from jax.experimental import pallas as pl
from jax.experimental.pallas import tpu as pltpu
from jax.experimental.pallas import tpu_sc as plsc

N_NODES_ = 29
E_RAW = 232
E_PAD = 240
N_CHUNKS = E_PAD // 16


def _rsqrt_newton(x):
    i = plsc.bitcast(x, jnp.int32)
    i = jnp.int32(0x5F3759DF) - lax.shift_right_arithmetic(i, jnp.int32(1))
    y = plsc.bitcast(i, jnp.float32)
    for _ in range(4):
        y = y * (1.5 - 0.5 * x * y * y)
    return y


def _sc_build_operators(src_pad, dst_pad, w1b, w2b):
    mesh = plsc.VectorSubcoreMesh(core_axis_name="c", subcore_axis_name="s")

    @functools.partial(
        pl.kernel,
        mesh=mesh,
        compiler_params=pltpu.CompilerParams(needs_layout_passes=False),
        out_type=(
            jax.ShapeDtypeStruct((87, 64), jnp.float32),
            jax.ShapeDtypeStruct((64, 32), jnp.float32),
        ),
        scratch_types=[
            pltpu.VMEM((E_PAD,), jnp.int32),
            pltpu.VMEM((E_PAD,), jnp.int32),
            pltpu.VMEM((6, 16), jnp.float32),
            pltpu.VMEM((2, 16), jnp.float32),
            pltpu.VMEM((32,), jnp.float32),
            pltpu.VMEM((32,), jnp.float32),
            pltpu.VMEM((87, 64), jnp.float32),
            pltpu.VMEM((64, 32), jnp.float32),
        ],
    )
    def k(src_hbm, dst_hbm, w1_hbm, w2_hbm, m1_hbm, m2_hbm,
          sv, dv, w1v, w2v, deg, dinv, m1v, m2v):
        cid = lax.axis_index("c")
        sid = lax.axis_index("s")

        @pl.when((cid == 0) & (sid == 0))
        def _():
            pltpu.sync_copy(src_hbm, sv)
            pltpu.sync_copy(dst_hbm, dv)
            pltpu.sync_copy(w1_hbm, w1v)
            pltpu.sync_copy(w2_hbm, w2v)

            lane = lax.iota(jnp.int32, 16)
            zeros = jnp.zeros((16,), jnp.float32)
            ones = jnp.ones((16,), jnp.float32)

            deg[pl.ds(0, 16)] = zeros
            deg[pl.ds(16, 16)] = zeros

            def zero_m1(r, carry):
                for col in range(0, 64, 16):
                    m1v[r, pl.ds(col, 16)] = zeros
                return carry

            lax.fori_loop(0, 87, zero_m1, 0)

            def zero_m2(r, carry):
                for col in range(0, 32, 16):
                    m2v[r, pl.ds(col, 16)] = zeros
                return carry

            lax.fori_loop(0, 64, zero_m2, 0)

            def deg_body(c, carry):
                dvec = dv[pl.ds(c * 16, 16)]
                valid = (c * 16 + lane) < E_RAW
                for j in range(16):
                    plsc.addupdate_scatter(
                        deg, [dvec], ones, mask=valid & (lane == j))
                return carry

            lax.fori_loop(0, N_CHUNKS, deg_body, 0)

            deg[pl.ds(0, 16)] = deg[pl.ds(0, 16)] + 1.0
            tail = jnp.where(lane < (N_NODES_ - 16), 1.0, 0.0)
            deg[pl.ds(16, 16)] = deg[pl.ds(16, 16)] + tail

            dinv[pl.ds(0, 16)] = _rsqrt_newton(deg[pl.ds(0, 16)])
            dinv[pl.ds(16, 16)] = _rsqrt_newton(
                jnp.maximum(deg[pl.ds(16, 16)], 1.0))

            w1vecs = [w1v[q, pl.ds(0, 16)] for q in range(6)]
            w2vecs = [w2v[q, pl.ds(0, 16)] for q in range(2)]

            def edge_body(c, carry):
                svec = sv[pl.ds(c * 16, 16)]
                dvec = dv[pl.ds(c * 16, 16)]
                nrm = (plsc.load_gather(dinv, [svec]) *
                       plsc.load_gather(dinv, [dvec]))
                valid = (c * 16 + lane) < E_RAW
                for f in range(3):
                    rows = svec * 3 + f
                    for o in range(2):
                        cols = dvec * 2 + o
                        val = nrm * w1vecs[f * 2 + o]
                        for j in range(16):
                            plsc.addupdate_scatter(
                                m1v, [rows, cols], val,
                                mask=valid & (lane == j))
                for o in range(2):
                    rows2 = svec * 2 + o
                    val2 = nrm * w2vecs[o]
                    for j in range(16):
                        plsc.addupdate_scatter(
                            m2v, [rows2, dvec], val2,
                            mask=valid & (lane == j))
                return carry

            lax.fori_loop(0, N_CHUNKS, edge_body, 0)

            for c in range(2):
                ids = lane + c * 16
                m = ids < N_NODES_
                dvv = dinv[pl.ds(c * 16, 16)]
                diag = dvv * dvv
                for f in range(3):
                    for o in range(2):
                        plsc.addupdate_scatter(
                            m1v, [ids * 3 + f, ids * 2 + o],
                            diag * w1vecs[f * 2 + o], mask=m)
                for o in range(2):
                    plsc.addupdate_scatter(
                        m2v, [ids * 2 + o, ids], diag * w2vecs[o], mask=m)

            pltpu.sync_copy(m1v, m1_hbm)
            pltpu.sync_copy(m2v, m2_hbm)

    return k(src_pad, dst_pad, w1b, w2b)


def _tc_body(x_ref, m1_ref, m2_ref, b1_ref, b2_ref, wf1_ref, bf1_ref,
             wf2_ref, bf2_ref, wf_ref, bf_ref, out_ref):
    dot = functools.partial(jnp.dot, preferred_element_type=jnp.float32)
    relu = lambda v: jnp.maximum(v, 0.0)
    x3 = x_ref[...]
    x = x3.reshape(x3.shape[0], 87)
    h = relu(dot(x, m1_ref[...]) + b1_ref[...])
    h = relu(dot(h, m2_ref[...]) + b2_ref[...])
    h = h[:, :N_NODES_]
    h = relu(dot(h, wf1_ref[...]) + bf1_ref[...])
    h = relu(dot(h, wf2_ref[...]) + bf2_ref[...])
    out_ref[...] = dot(h, wf_ref[...]) + bf_ref[...]


def _dense_chain(feature, m1, m2, b1f, b2f, wf1, bf1, wf2, bf2, wf, bf,
                 block_b):
    b_total = feature.shape[0]
    grid = (b_total // block_b,)
    vfull = lambda shape: pl.BlockSpec(shape, lambda i: tuple(0 for _ in shape))
    return pl.pallas_call(
        _tc_body,
        grid=grid,
        in_specs=[
            pl.BlockSpec((block_b, N_NODES_, 3), lambda i: (i, 0, 0)),
            vfull((87, 64)),
            vfull((64, 32)),
            vfull((1, 64)),
            vfull((1, 32)),
            vfull((29, 128)),
            vfull((1, 128)),
            vfull((128, 128)),
            vfull((1, 128)),
            vfull((128, 1296)),
            vfull((1, 1296)),
        ],
        out_specs=pl.BlockSpec((block_b, 1296), lambda i: (i, 0)),
        out_shape=jax.ShapeDtypeStruct((b_total, 1296), jnp.float32),
        compiler_params=pltpu.CompilerParams(
            dimension_semantics=("arbitrary",)),
    )(feature, m1, m2, b1f, b2f, wf1, bf1, wf2, bf2, wf, bf)


def kernel(feature, edge_index, W1, b1, W2, b2, Wf1, bf1, Wf2, bf2, Wf, bf):
    src_pad = jnp.pad(edge_index[0], (0, E_PAD - E_RAW)).astype(jnp.int32)
    dst_pad = jnp.pad(edge_index[1], (0, E_PAD - E_RAW)).astype(jnp.int32)
    w1b = jnp.broadcast_to(W1.reshape(6, 1), (6, 16)).astype(jnp.float32)
    w2b = jnp.broadcast_to(W2.reshape(2, 1), (2, 16)).astype(jnp.float32)
    m1, m2 = _sc_build_operators(src_pad, dst_pad, w1b, w2b)

    b1f = jnp.pad(jnp.tile(b1, N_NODES_), (0, 6)).reshape(1, 64)
    b2f = jnp.pad(jnp.broadcast_to(b2, (N_NODES_,)), (0, 3)).reshape(1, 32)

    return _dense_chain(
        feature, m1, m2, b1f, b2f,
        Wf1, bf1.reshape(1, 128), Wf2, bf2.reshape(1, 128),
        Wf, bf.reshape(1, 1296), block_b=512)


# speedup vs baseline: 2.3048x; 2.3048x over previous
import functools

import jax
import jax.numpy as jnp
from jax import lax
from jax.experimental import pallas as pl
from jax.experimental.pallas import tpu as pltpu
from jax.experimental.pallas import tpu_sc as plsc

N_NODES_ = 29
E_RAW = 232
E_PAD = 240
N_CHUNKS = E_PAD // 16


def _rsqrt_newton(x):
    i = plsc.bitcast(x, jnp.int32)
    i = jnp.int32(0x5F3759DF) - lax.shift_right_arithmetic(i, jnp.int32(1))
    y = plsc.bitcast(i, jnp.float32)
    for _ in range(4):
        y = y * (1.5 - 0.5 * x * y * y)
    return y


def _sc_build_operators(src_pad, dst_pad, w1b, w2b):
    mesh = plsc.VectorSubcoreMesh(core_axis_name="c", subcore_axis_name="s")

    @functools.partial(
        pl.kernel,
        mesh=mesh,
        compiler_params=pltpu.CompilerParams(needs_layout_passes=False),
        out_type=(
            jax.ShapeDtypeStruct((87, 64), jnp.float32),
            jax.ShapeDtypeStruct((64, 32), jnp.float32),
        ),
        scratch_types=[
            pltpu.VMEM((E_PAD,), jnp.int32),
            pltpu.VMEM((E_PAD,), jnp.int32),
            pltpu.VMEM((6, 16), jnp.float32),
            pltpu.VMEM((2, 16), jnp.float32),
            pltpu.VMEM((32,), jnp.float32),
            pltpu.VMEM((32,), jnp.float32),
            pltpu.VMEM((87, 64), jnp.float32),
            pltpu.VMEM((64, 32), jnp.float32),
        ],
    )
    def k(src_hbm, dst_hbm, w1_hbm, w2_hbm, m1_hbm, m2_hbm,
          sv, dv, w1v, w2v, deg, dinv, m1v, m2v):
        cid = lax.axis_index("c")
        sid = lax.axis_index("s")

        @pl.when((cid == 0) & (sid == 0))
        def _():
            pltpu.sync_copy(src_hbm, sv)
            pltpu.sync_copy(dst_hbm, dv)
            pltpu.sync_copy(w1_hbm, w1v)
            pltpu.sync_copy(w2_hbm, w2v)

            lane = lax.iota(jnp.int32, 16)
            zeros = jnp.zeros((16,), jnp.float32)
            ones = jnp.ones((16,), jnp.float32)

            deg[pl.ds(0, 16)] = zeros
            deg[pl.ds(16, 16)] = zeros

            def zero_m1(r, carry):
                for col in range(0, 64, 16):
                    m1v[r, pl.ds(col, 16)] = zeros
                return carry

            lax.fori_loop(0, 87, zero_m1, 0)

            def zero_m2(r, carry):
                for col in range(0, 32, 16):
                    m2v[r, pl.ds(col, 16)] = zeros
                return carry

            lax.fori_loop(0, 64, zero_m2, 0)

            def deg_body(c, carry):
                dvec = dv[pl.ds(c * 16, 16)]
                valid = (c * 16 + lane) < E_RAW
                for j in range(16):
                    plsc.addupdate_scatter(
                        deg, [dvec], ones, mask=valid & (lane == j))
                return carry

            lax.fori_loop(0, N_CHUNKS, deg_body, 0)

            deg[pl.ds(0, 16)] = deg[pl.ds(0, 16)] + 1.0
            tail = jnp.where(lane < (N_NODES_ - 16), 1.0, 0.0)
            deg[pl.ds(16, 16)] = deg[pl.ds(16, 16)] + tail

            dinv[pl.ds(0, 16)] = _rsqrt_newton(deg[pl.ds(0, 16)])
            dinv[pl.ds(16, 16)] = _rsqrt_newton(
                jnp.maximum(deg[pl.ds(16, 16)], 1.0))

            w1vecs = [w1v[q, pl.ds(0, 16)] for q in range(6)]
            w2vecs = [w2v[q, pl.ds(0, 16)] for q in range(2)]

            def edge_body(c, carry):
                svec = sv[pl.ds(c * 16, 16)]
                dvec = dv[pl.ds(c * 16, 16)]
                nrm = (plsc.load_gather(dinv, [svec]) *
                       plsc.load_gather(dinv, [dvec]))
                valid = (c * 16 + lane) < E_RAW
                for f in range(3):
                    rows = svec * 3 + f
                    for o in range(2):
                        cols = dvec * 2 + o
                        val = nrm * w1vecs[f * 2 + o]
                        for j in range(16):
                            plsc.addupdate_scatter(
                                m1v, [rows, cols], val,
                                mask=valid & (lane == j))
                for o in range(2):
                    rows2 = svec * 2 + o
                    val2 = nrm * w2vecs[o]
                    for j in range(16):
                        plsc.addupdate_scatter(
                            m2v, [rows2, dvec], val2,
                            mask=valid & (lane == j))
                return carry

            lax.fori_loop(0, N_CHUNKS, edge_body, 0)

            for c in range(2):
                ids = lane + c * 16
                m = ids < N_NODES_
                dvv = dinv[pl.ds(c * 16, 16)]
                diag = dvv * dvv
                for f in range(3):
                    for o in range(2):
                        plsc.addupdate_scatter(
                            m1v, [ids * 3 + f, ids * 2 + o],
                            diag * w1vecs[f * 2 + o], mask=m)
                for o in range(2):
                    plsc.addupdate_scatter(
                        m2v, [ids * 2 + o, ids], diag * w2vecs[o], mask=m)

            pltpu.sync_copy(m1v, m1_hbm)
            pltpu.sync_copy(m2v, m2_hbm)

    return k(src_pad, dst_pad, w1b, w2b)


def _tc_body(x_ref, m1_ref, m2_ref, b1_ref, b2_ref, wf1_ref, bf1_ref,
             wf2_ref, bf2_ref, wf_ref, bf_ref, out_ref):
    dot = functools.partial(jnp.dot, preferred_element_type=jnp.float32)
    relu = lambda v: jnp.maximum(v, 0.0)
    x = x_ref[...]
    h = relu(dot(x, m1_ref[...]) + b1_ref[...])
    h = relu(dot(h, m2_ref[...]) + b2_ref[...])
    h = h[:, :N_NODES_]
    h = relu(dot(h, wf1_ref[...]) + bf1_ref[...])
    h = relu(dot(h, wf2_ref[...]) + bf2_ref[...])
    out_ref[...] = dot(h, wf_ref[...]) + bf_ref[...]


def _dense_chain(x, m1, m2, b1f, b2f, wf1, bf1, wf2, bf2, wf, bf,
                 block_b):
    b_total = x.shape[0]
    grid = (b_total // block_b,)
    vfull = lambda shape: pl.BlockSpec(shape, lambda i: tuple(0 for _ in shape))
    return pl.pallas_call(
        _tc_body,
        grid=grid,
        in_specs=[
            pl.BlockSpec((block_b, 87), lambda i: (i, 0)),
            vfull((87, 64)),
            vfull((64, 32)),
            vfull((1, 64)),
            vfull((1, 32)),
            vfull((29, 128)),
            vfull((1, 128)),
            vfull((128, 128)),
            vfull((1, 128)),
            vfull((128, 1296)),
            vfull((1, 1296)),
        ],
        out_specs=pl.BlockSpec((block_b, 1296), lambda i: (i, 0)),
        out_shape=jax.ShapeDtypeStruct((b_total, 1296), jnp.float32),
        compiler_params=pltpu.CompilerParams(
            dimension_semantics=("arbitrary",)),
    )(x, m1, m2, b1f, b2f, wf1, bf1, wf2, bf2, wf, bf)


def kernel(feature, edge_index, W1, b1, W2, b2, Wf1, bf1, Wf2, bf2, Wf, bf):
    src_pad = jnp.pad(edge_index[0], (0, E_PAD - E_RAW)).astype(jnp.int32)
    dst_pad = jnp.pad(edge_index[1], (0, E_PAD - E_RAW)).astype(jnp.int32)
    w1b = jnp.broadcast_to(W1.reshape(6, 1), (6, 16)).astype(jnp.float32)
    w2b = jnp.broadcast_to(W2.reshape(2, 1), (2, 16)).astype(jnp.float32)
    m1, m2 = _sc_build_operators(src_pad, dst_pad, w1b, w2b)

    b1f = jnp.pad(jnp.tile(b1, N_NODES_), (0, 6)).reshape(1, 64)
    b2f = jnp.pad(jnp.broadcast_to(b2, (N_NODES_,)), (0, 3)).reshape(1, 32)

    x = feature.reshape(feature.shape[0], 87)
    return _dense_chain(
        x, m1, m2, b1f, b2f,
        Wf1, bf1.reshape(1, 128), Wf2, bf2.reshape(1, 128),
        Wf, bf.reshape(1, 1296), block_b=1024)


# speedup vs baseline: 5.9092x; 2.5639x over previous
import functools

import jax
import jax.numpy as jnp
from jax import lax
from jax.experimental import pallas as pl
from jax.experimental.pallas import tpu as pltpu
from jax.experimental.pallas import tpu_sc as plsc

N_NODES_ = 29
E_RAW = 232
E_PAD = 240
N_CHUNKS = E_PAD // 16


def _rsqrt_newton(x):
    i = plsc.bitcast(x, jnp.int32)
    i = jnp.int32(0x5F3759DF) - lax.shift_right_arithmetic(i, jnp.int32(1))
    y = plsc.bitcast(i, jnp.float32)
    for _ in range(4):
        y = y * (1.5 - 0.5 * x * y * y)
    return y


def _sc_build_adj(src_pad, dst_pad):
    mesh = plsc.VectorSubcoreMesh(core_axis_name="c", subcore_axis_name="s")

    @functools.partial(
        pl.kernel,
        mesh=mesh,
        compiler_params=pltpu.CompilerParams(needs_layout_passes=False),
        out_type=jax.ShapeDtypeStruct((32, 32), jnp.float32),
        scratch_types=[
            pltpu.VMEM((E_PAD,), jnp.int32),
            pltpu.VMEM((E_PAD,), jnp.int32),
            pltpu.VMEM((32,), jnp.float32),
            pltpu.VMEM((32,), jnp.float32),
            pltpu.VMEM((32, 32), jnp.float32),
        ],
    )
    def k(src_hbm, dst_hbm, ah_hbm, sv, dv, deg, dinv, ahv):
        cid = lax.axis_index("c")
        sid = lax.axis_index("s")

        @pl.when((cid == 0) & (sid == 0))
        def _():
            pltpu.sync_copy(src_hbm, sv)
            pltpu.sync_copy(dst_hbm, dv)

            lane = lax.iota(jnp.int32, 16)
            zeros = jnp.zeros((16,), jnp.float32)
            ones = jnp.ones((16,), jnp.float32)

            deg[pl.ds(0, 16)] = zeros
            deg[pl.ds(16, 16)] = zeros

            def zero_ah(r, carry):
                ahv[r, pl.ds(0, 16)] = zeros
                ahv[r, pl.ds(16, 16)] = zeros
                return carry

            lax.fori_loop(0, 32, zero_ah, 0)

            def deg_body(c, carry):
                dvec = dv[pl.ds(c * 16, 16)]
                valid = (c * 16 + lane) < E_RAW
                for j in range(16):
                    plsc.addupdate_scatter(
                        deg, [dvec], ones, mask=valid & (lane == j))
                return carry

            lax.fori_loop(0, N_CHUNKS, deg_body, 0)

            deg[pl.ds(0, 16)] = deg[pl.ds(0, 16)] + 1.0
            tail = jnp.where(lane < (N_NODES_ - 16), 1.0, 0.0)
            deg[pl.ds(16, 16)] = deg[pl.ds(16, 16)] + tail

            dinv[pl.ds(0, 16)] = _rsqrt_newton(deg[pl.ds(0, 16)])
            dinv[pl.ds(16, 16)] = _rsqrt_newton(
                jnp.maximum(deg[pl.ds(16, 16)], 1.0))

            def edge_body(c, carry):
                svec = sv[pl.ds(c * 16, 16)]
                dvec = dv[pl.ds(c * 16, 16)]
                nrm = (plsc.load_gather(dinv, [svec]) *
                       plsc.load_gather(dinv, [dvec]))
                valid = (c * 16 + lane) < E_RAW
                for j in range(16):
                    plsc.addupdate_scatter(
                        ahv, [dvec, svec], nrm, mask=valid & (lane == j))
                return carry

            lax.fori_loop(0, N_CHUNKS, edge_body, 0)

            for c in range(2):
                ids = lane + c * 16
                dvv = dinv[pl.ds(c * 16, 16)]
                plsc.addupdate_scatter(
                    ahv, [ids, ids], dvv * dvv, mask=ids < N_NODES_)

            pltpu.sync_copy(ahv, ah_hbm)

    return k(src_pad, dst_pad)


def _tc_body(ft_ref, ah_ref, w1_ref, b1_ref, w2_ref, b2_ref, wf1t_ref,
             bf1_ref, wf2t_ref, bf2_ref, wft_ref, bf_ref, outt_ref):
    dot = functools.partial(jnp.dot, preferred_element_type=jnp.float32)
    relu = lambda v: jnp.maximum(v, 0.0)
    ft = ft_ref[...]
    ah = ah_ref[...]
    ah29 = ah[:, :N_NODES_]
    f0 = ft[0]
    f1 = ft[1]
    f2 = ft[2]
    s0 = f0 * w1_ref[0, 0] + f1 * w1_ref[1, 0] + f2 * w1_ref[2, 0]
    s1 = f0 * w1_ref[0, 1] + f1 * w1_ref[1, 1] + f2 * w1_ref[2, 1]
    h10 = relu(dot(ah29, s0) + b1_ref[0])
    h11 = relu(dot(ah29, s1) + b1_ref[1])
    t = h10 * w2_ref[0, 0] + h11 * w2_ref[1, 0]
    h2 = relu(dot(ah, t) + b2_ref[0])[:N_NODES_]
    h3 = relu(dot(wf1t_ref[...], h2) + bf1_ref[...])
    h4 = relu(dot(wf2t_ref[...], h3) + bf2_ref[...])
    outt_ref[...] = dot(wft_ref[...], h4) + bf_ref[...]


def _dense_chain_t(ft, ah, w1, b1, w2, b2, wf1t, bf1c, wf2t, bf2c, wft, bfc,
                   block_b):
    b_total = ft.shape[2]
    grid = (b_total // block_b,)
    vfull = lambda shape: pl.BlockSpec(shape, lambda i: tuple(0 for _ in shape))
    sfull = lambda shape: pl.BlockSpec(
        shape, lambda i: tuple(0 for _ in shape), memory_space=pltpu.SMEM)
    return pl.pallas_call(
        _tc_body,
        grid=grid,
        in_specs=[
            pl.BlockSpec((3, N_NODES_, block_b), lambda i: (0, 0, i)),
            vfull((32, 32)),
            sfull((3, 2)),
            sfull((2,)),
            sfull((2, 1)),
            sfull((1,)),
            vfull((128, 29)),
            vfull((128, 1)),
            vfull((128, 128)),
            vfull((128, 1)),
            vfull((1296, 128)),
            vfull((1296, 1)),
        ],
        out_specs=pl.BlockSpec((1296, block_b), lambda i: (0, i)),
        out_shape=jax.ShapeDtypeStruct((1296, b_total), jnp.float32),
        compiler_params=pltpu.CompilerParams(
            dimension_semantics=("arbitrary",)),
    )(ft, ah, w1, b1, w2, b2, wf1t, bf1c, wf2t, bf2c, wft, bfc)


def kernel(feature, edge_index, W1, b1, W2, b2, Wf1, bf1, Wf2, bf2, Wf, bf):
    src_pad = jnp.pad(edge_index[0], (0, E_PAD - E_RAW)).astype(jnp.int32)
    dst_pad = jnp.pad(edge_index[1], (0, E_PAD - E_RAW)).astype(jnp.int32)
    ah = _sc_build_adj(src_pad, dst_pad)

    ft = jnp.transpose(feature, (2, 1, 0))
    outt = _dense_chain_t(
        ft, ah, W1, b1, W2, b2,
        Wf1.T, bf1.reshape(128, 1), Wf2.T, bf2.reshape(128, 1),
        Wf.T, bf.reshape(1296, 1), block_b=1024)
    return outt.T
